# Initial kernel scaffold; baseline (speedup 1.0000x reference)
import jax
import jax.numpy as jnp
from jax.experimental import pallas as pl


def kernel(servers, users, masks, Wemb, bemb, Wq, Wk, Wv, Wo, bo):
    raise NotImplementedError("write your pallas kernel here")


# speedup vs baseline: 14.1548x; 14.1548x over previous
import jax
import jax.numpy as jnp
from jax.experimental import pallas as pl
from jax.experimental.pallas import tpu as pltpu

N_USERS = 5000
N_SERVERS = 1000
EMBED_DIM = 128
N_HEADS = 8

_UPAD = 5120
_SPAD = 1024
_UROWS = _UPAD // 128
_SROWS = _SPAD // 128
_NEG = float("-inf")


def _attention(x, Wemb, bemb, Wq, Wk, Wv, Wo, bo):
    h = x @ Wemb + bemb
    N = h.shape[0]
    dh = EMBED_DIM // N_HEADS
    q = (h @ Wq).reshape(N, N_HEADS, dh).transpose(1, 0, 2)
    k = (h @ Wk).reshape(N, N_HEADS, dh).transpose(1, 0, 2)
    v = (h @ Wv).reshape(N, N_HEADS, dh).transpose(1, 0, 2)
    scores = (q @ k.transpose(0, 2, 1)) / jnp.sqrt(jnp.float32(dh))
    attn = jax.nn.softmax(scores, axis=-1)
    out = (attn @ v).transpose(1, 0, 2).reshape(N, EMBED_DIM)
    return out @ Wo + bo


def _greedy_kernel(uscore_ref, sscore_ref, masks_ref, cap_in_ref, wl_ref,
                   alloc_ref, usage_ref,
                   us_s, cap0_s, cap1_s, cap2_s, cap3_s):
    us_s[...] = uscore_ref[...]
    cap0_s[...] = cap_in_ref[0]
    cap1_s[...] = cap_in_ref[1]
    cap2_s[...] = cap_in_ref[2]
    cap3_s[...] = cap_in_ref[3]
    alloc_ref[...] = jnp.full((_UROWS, 128), -1.0, jnp.float32)
    usage_ref[...] = jnp.zeros((_SROWS, 128), jnp.float32)

    uiota = (jax.lax.broadcasted_iota(jnp.int32, (_UROWS, 128), 0) * 128
             + jax.lax.broadcasted_iota(jnp.int32, (_UROWS, 128), 1))
    siota = (jax.lax.broadcasted_iota(jnp.int32, (_SROWS, 128), 0) * 128
             + jax.lax.broadcasted_iota(jnp.int32, (_SROWS, 128), 1))
    sscore = sscore_ref[...]

    def step(_, carry):
        usv = us_s[...]
        um = jnp.max(usv)
        u = jnp.min(jnp.where(usv == um, uiota, jnp.int32(_UPAD)))
        us_s[...] = jnp.where(uiota == u, _NEG, usv)

        w0 = wl_ref[0, u]
        w1 = wl_ref[1, u]
        w2 = wl_ref[2, u]
        w3 = wl_ref[3, u]
        mrow = masks_ref[u]
        feas = (mrow & (cap0_s[...] >= w0) & (cap1_s[...] >= w1)
                & (cap2_s[...] >= w2) & (cap3_s[...] >= w3))
        msc = jnp.where(feas, sscore, _NEG)
        sm = jnp.max(msc)
        best = jnp.min(jnp.where(msc == sm, siota, jnp.int32(_SPAD)))
        valid = sm > -1.0
        validf = jnp.where(valid, jnp.float32(1.0), jnp.float32(0.0))
        oh = jnp.where(siota == best, validf, jnp.float32(0.0))
        cap0_s[...] = cap0_s[...] - w0 * oh
        cap1_s[...] = cap1_s[...] - w1 * oh
        cap2_s[...] = cap2_s[...] - w2 * oh
        cap3_s[...] = cap3_s[...] - w3 * oh
        usage_ref[...] = usage_ref[...] + oh
        aval = jnp.where(valid, best.astype(jnp.float32), jnp.float32(-1.0))
        alloc_ref[...] = jnp.where(uiota == u, aval, alloc_ref[...])
        return carry

    jax.lax.fori_loop(0, N_USERS, step, 0)


def kernel(servers, users, masks, Wemb, bemb, Wq, Wk, Wv, Wo, bo):
    context_vector = _attention(users[:, 2:], Wemb, bemb, Wq, Wk, Wv, Wo, bo)
    user_scores = jnp.mean(context_vector, axis=1)
    server_context_vector = _attention(servers[:, 3:], Wemb, bemb, Wq, Wk, Wv, Wo, bo)
    server_scores = jnp.mean(server_context_vector, axis=1)

    uscore_pad = jnp.full((_UPAD,), _NEG, jnp.float32).at[:N_USERS].set(
        user_scores).reshape(_UROWS, 128)
    sscore_pad = jnp.full((_SPAD,), _NEG, jnp.float32).at[:N_SERVERS].set(
        server_scores).reshape(_SROWS, 128)
    masks_pad = jnp.pad(masks, ((0, 0), (0, _SPAD - N_SERVERS))).reshape(
        N_USERS, _SROWS, 128)
    cap_pad = jnp.pad(servers[:, 3:], ((0, _SPAD - N_SERVERS), (0, 0))).T.reshape(
        4, _SROWS, 128)
    wl = users[:, 2:6].T

    alloc_pad, usage_pad = pl.pallas_call(
        _greedy_kernel,
        out_shape=(
            jax.ShapeDtypeStruct((_UROWS, 128), jnp.float32),
            jax.ShapeDtypeStruct((_SROWS, 128), jnp.float32),
        ),
        in_specs=[
            pl.BlockSpec(memory_space=pltpu.VMEM),
            pl.BlockSpec(memory_space=pltpu.VMEM),
            pl.BlockSpec(memory_space=pltpu.VMEM),
            pl.BlockSpec(memory_space=pltpu.VMEM),
            pl.BlockSpec(memory_space=pltpu.SMEM),
        ],
        out_specs=(
            pl.BlockSpec(memory_space=pltpu.VMEM),
            pl.BlockSpec(memory_space=pltpu.VMEM),
        ),
        scratch_shapes=[
            pltpu.VMEM((_UROWS, 128), jnp.float32),
            pltpu.VMEM((_SROWS, 128), jnp.float32),
            pltpu.VMEM((_SROWS, 128), jnp.float32),
            pltpu.VMEM((_SROWS, 128), jnp.float32),
            pltpu.VMEM((_SROWS, 128), jnp.float32),
        ],
    )(uscore_pad, sscore_pad, masks_pad, cap_pad, wl)

    alloc = alloc_pad.reshape(-1)[:N_USERS]
    usage = usage_pad.reshape(-1)[:N_SERVERS]

    allocated_user_num = jnp.sum(alloc != -1.0)
    user_allocated_prop = allocated_user_num.astype(jnp.float32) / N_USERS
    used_server_num = jnp.count_nonzero(usage)
    server_used_prop = used_server_num.astype(jnp.float32) / N_SERVERS
    sp = jax.nn.softplus
    total_loss = 2.0 * (sp(-user_allocated_prop) + sp(server_used_prop))
    loss = (jnp.sum(jax.nn.softmax(context_vector, axis=0) * total_loss)
            + jnp.sum(jax.nn.softmax(server_context_vector, axis=0) * total_loss))
    return (loss, alloc, usage, user_allocated_prop, server_used_prop)
